# baseline (speedup 1.0000x reference)
import functools

import jax
import jax.numpy as jnp
from jax import lax
from jax.experimental import pallas as pl
from jax.experimental.pallas import tpu as pltpu
from jax.experimental.pallas import tpu_sc as plsc

B = 16384
D = 64
N = 1000000
BLK = 1024
K = 64
NBLK = (N + BLK - 1) // BLK
NPG = NBLK * (K // 2)

NC = 2
NS = 16
NW = NC * NS
BPW = B // NW
CH = BPW // 2


def _slots(idx):
    blk = idx >> 10
    order = jnp.argsort(blk, stable=True)
    bs = blk[order]
    first = jnp.searchsorted(bs, bs, side="left")
    rank_s = jnp.arange(B, dtype=jnp.int32) - first.astype(jnp.int32)
    rank = jnp.zeros((B,), jnp.int32).at[order].set(rank_s)
    rank = jnp.minimum(rank, K - 1)
    pos = jnp.full((NBLK, K), -1, jnp.int32).at[blk, rank].set(
        idx & (BLK - 1), mode="drop")
    gq = blk * (K // 2) + (rank & (K // 2 - 1))
    par = rank >> 5
    return pos, gq, par


def _sel_body(pos_ref, tt_ref, out_ref):
    r = pl.program_id(0)
    chunk = tt_ref[...]
    col = r * BLK + lax.broadcasted_iota(jnp.int32, (D, BLK), 1)
    chunk = jnp.where(col < N, chunk, 0.0)
    p = pos_ref[0]
    iota = lax.broadcasted_iota(jnp.int32, (BLK, K), 0)
    onehot = (iota == p).astype(jnp.float32)
    sd = lax.dot_general(onehot, chunk, (((0,), (1,)), ((), ())),
                         preferred_element_type=jnp.float32)
    out_ref[...] = jnp.concatenate([sd[:K // 2], sd[K // 2:]], axis=1)


def _select_pack(tt, pos):
    return pl.pallas_call(
        _sel_body,
        grid=(NBLK,),
        in_specs=[
            pl.BlockSpec((1, 1, K), lambda r: (r, 0, 0)),
            pl.BlockSpec((D, BLK), lambda r: (0, r)),
        ],
        out_specs=pl.BlockSpec((K // 2, 128), lambda r: (r, 0)),
        out_shape=jax.ShapeDtypeStruct((NPG, 128), jnp.float32),
    )(pos.reshape(NBLK, 1, K), tt)


def _sc_gather_body(idx_hbm, tab, out, idx_v, buf0, buf1, sg0, sg1, so0, so1):
    wid = lax.axis_index("s") * NC + lax.axis_index("c")
    base = wid * BPW
    pltpu.sync_copy(idx_hbm.at[pl.ds(base, BPW)], idx_v)
    g0 = pltpu.async_copy(tab.at[idx_v.at[pl.ds(0, CH)]], buf0, sg0)
    g1 = pltpu.async_copy(tab.at[idx_v.at[pl.ds(CH, CH)]], buf1, sg1)
    g0.wait()
    o0 = pltpu.async_copy(buf0, out.at[pl.ds(base, CH)], so0)
    g1.wait()
    o1 = pltpu.async_copy(buf1, out.at[pl.ds(base + CH, CH)], so1)
    o0.wait()
    o1.wait()


@functools.cache
def _sc_gather():
    mesh = plsc.VectorSubcoreMesh(
        core_axis_name="c", subcore_axis_name="s", num_cores=NC, num_subcores=NS
    )
    return pl.kernel(
        _sc_gather_body,
        out_type=jax.ShapeDtypeStruct((B, 128), jnp.float32),
        mesh=mesh,
        scratch_types=[
            pltpu.VMEM((BPW,), jnp.int32),
            pltpu.VMEM((CH, 128), jnp.float32),
            pltpu.VMEM((CH, 128), jnp.float32),
            pltpu.SemaphoreType.DMA,
            pltpu.SemaphoreType.DMA,
            pltpu.SemaphoreType.DMA,
            pltpu.SemaphoreType.DMA,
        ],
    )


def _half(x, par):
    return x[:, :D] * (1.0 - par) + x[:, D:] * par


def _tc_body(up_ref, ip_ref, gu_ref, gi_ref, mu_ref, mi_ref,
             w1_ref, b1_ref, w2_ref, b2_ref, w3_ref, b3_ref,
             wp_ref, bp_ref, out_ref):
    upar = up_ref[...].astype(jnp.float32)
    ipar = ip_ref[...].astype(jnp.float32)
    mu = _half(mu_ref[...], upar)
    mi = _half(mi_ref[...], ipar)
    gu = _half(gu_ref[...], upar)
    gi = _half(gi_ref[...], ipar)
    w1 = w1_ref[...]
    h = jnp.dot(mu, w1[:D], preferred_element_type=jnp.float32)
    h = h + jnp.dot(mi, w1[D:], preferred_element_type=jnp.float32)
    h = jnp.maximum(h + b1_ref[...], 0.0)
    h = jnp.maximum(
        jnp.dot(h, w2_ref[...], preferred_element_type=jnp.float32) + b2_ref[...], 0.0)
    h = jnp.maximum(
        jnp.dot(h, w3_ref[...], preferred_element_type=jnp.float32) + b3_ref[...], 0.0)
    g = gu * gi
    wp = wp_ref[...]
    pred = jnp.dot(g, wp[:D], preferred_element_type=jnp.float32)
    pred = pred + jnp.dot(h, wp[D:], preferred_element_type=jnp.float32)
    out_ref[...] = pred + bp_ref[...]


def _tc_dense(upar, ipar, gu, gi, mu, mi, W1, b1, W2, b2, W3, b3, Wp, bp):
    R = 2048
    grid = (B // R,)
    row_spec = pl.BlockSpec((R, 128), lambda r: (r, 0))
    par_spec = pl.BlockSpec((R, 1), lambda r: (r, 0))

    def full(shape):
        return pl.BlockSpec(shape, lambda r: (0,) * len(shape))

    return pl.pallas_call(
        _tc_body,
        grid=grid,
        in_specs=[
            par_spec, par_spec,
            row_spec, row_spec, row_spec, row_spec,
            full(W1.shape), full((1, b1.shape[0])),
            full(W2.shape), full((1, b2.shape[0])),
            full(W3.shape), full((1, b3.shape[0])),
            full(Wp.shape), full((1, 1)),
        ],
        out_specs=pl.BlockSpec((R, 1), lambda r: (r, 0)),
        out_shape=jax.ShapeDtypeStruct((B, 1), jnp.float32),
    )(upar, ipar, gu, gi, mu, mi, W1, b1.reshape(1, -1), W2, b2.reshape(1, -1),
      W3, b3.reshape(1, -1), Wp, bp.reshape(1, 1))


def kernel(u, i, gmf_user_table, gmf_item_table, mlp_user_table, mlp_item_table,
           W1, b1, W2, b2, W3, b3, Wp, bp):
    u = u.astype(jnp.int32)
    i = i.astype(jnp.int32)
    pos_u, gq_u, par_u = _slots(u)
    pos_i, gq_i, par_i = _slots(i)
    gather = _sc_gather()
    packed = [
        _select_pack(jnp.swapaxes(t, 0, 1), p)
        for t, p in (
            (gmf_user_table, pos_u), (gmf_item_table, pos_i),
            (mlp_user_table, pos_u), (mlp_item_table, pos_i),
        )
    ]
    gu = gather(gq_u, packed[0])
    gi = gather(gq_i, packed[1])
    mu = gather(gq_u, packed[2])
    mi = gather(gq_i, packed[3])
    out = _tc_dense(par_u.reshape(B, 1), par_i.reshape(B, 1),
                    gu, gi, mu, mi, W1, b1, W2, b2, W3, b3, Wp, bp)
    return out[:, 0]


# speedup vs baseline: 1.6648x; 1.6648x over previous
import functools

import jax
import jax.numpy as jnp
from jax import lax
from jax.experimental import pallas as pl
from jax.experimental.pallas import tpu as pltpu
from jax.experimental.pallas import tpu_sc as plsc

B = 16384
D = 64
N = 1000000

NC = 2
NS = 16
NW = NC * NS
BPW = B // NW
CH = BPW // 2


def _sc_gather_body(idx_hbm, tab, out, idx_v, buf0, buf1, sg0, sg1, so0, so1):
    wid = lax.axis_index("s") * NC + lax.axis_index("c")
    base = wid * BPW
    pltpu.sync_copy(idx_hbm.at[pl.ds(base, BPW)], idx_v)
    g0 = pltpu.async_copy(tab.at[idx_v.at[pl.ds(0, CH)]], buf0, sg0)
    g1 = pltpu.async_copy(tab.at[idx_v.at[pl.ds(CH, CH)]], buf1, sg1)
    g0.wait()
    o0 = pltpu.async_copy(buf0, out.at[pl.ds(base, CH)], so0)
    g1.wait()
    o1 = pltpu.async_copy(buf1, out.at[pl.ds(base + CH, CH)], so1)
    o0.wait()
    o1.wait()


@functools.cache
def _sc_gather():
    mesh = plsc.VectorSubcoreMesh(
        core_axis_name="c", subcore_axis_name="s", num_cores=NC, num_subcores=NS
    )
    return pl.kernel(
        _sc_gather_body,
        out_type=jax.ShapeDtypeStruct((B, 128), jnp.float32),
        mesh=mesh,
        scratch_types=[
            pltpu.VMEM((BPW,), jnp.int32),
            pltpu.VMEM((CH, 128), jnp.float32),
            pltpu.VMEM((CH, 128), jnp.float32),
            pltpu.SemaphoreType.DMA,
            pltpu.SemaphoreType.DMA,
            pltpu.SemaphoreType.DMA,
            pltpu.SemaphoreType.DMA,
        ],
    )


def _half(x, par):
    return x[:, :D] * (1.0 - par) + x[:, D:] * par


def _tc_body(up_ref, ip_ref, gu_ref, gi_ref, mu_ref, mi_ref,
             w1_ref, b1_ref, w2_ref, b2_ref, w3_ref, b3_ref,
             wp_ref, bp_ref, out_ref):
    upar = up_ref[...].astype(jnp.float32)
    ipar = ip_ref[...].astype(jnp.float32)
    mu = _half(mu_ref[...], upar)
    mi = _half(mi_ref[...], ipar)
    gu = _half(gu_ref[...], upar)
    gi = _half(gi_ref[...], ipar)
    w1 = w1_ref[...]
    h = jnp.dot(mu, w1[:D], preferred_element_type=jnp.float32)
    h = h + jnp.dot(mi, w1[D:], preferred_element_type=jnp.float32)
    h = jnp.maximum(h + b1_ref[...], 0.0)
    h = jnp.maximum(
        jnp.dot(h, w2_ref[...], preferred_element_type=jnp.float32) + b2_ref[...], 0.0)
    h = jnp.maximum(
        jnp.dot(h, w3_ref[...], preferred_element_type=jnp.float32) + b3_ref[...], 0.0)
    g = gu * gi
    wp = wp_ref[...]
    pred = jnp.dot(g, wp[:D], preferred_element_type=jnp.float32)
    pred = pred + jnp.dot(h, wp[D:], preferred_element_type=jnp.float32)
    out_ref[...] = pred + bp_ref[...]


def _tc_dense(upar, ipar, gu, gi, mu, mi, W1, b1, W2, b2, W3, b3, Wp, bp):
    R = 2048
    grid = (B // R,)
    row_spec = pl.BlockSpec((R, 128), lambda r: (r, 0))
    par_spec = pl.BlockSpec((R, 1), lambda r: (r, 0))

    def full(shape):
        return pl.BlockSpec(shape, lambda r: (0,) * len(shape))

    return pl.pallas_call(
        _tc_body,
        grid=grid,
        in_specs=[
            par_spec, par_spec,
            row_spec, row_spec, row_spec, row_spec,
            full(W1.shape), full((1, b1.shape[0])),
            full(W2.shape), full((1, b2.shape[0])),
            full(W3.shape), full((1, b3.shape[0])),
            full(Wp.shape), full((1, 1)),
        ],
        out_specs=pl.BlockSpec((R, 1), lambda r: (r, 0)),
        out_shape=jax.ShapeDtypeStruct((B, 1), jnp.float32),
    )(upar, ipar, gu, gi, mu, mi, W1, b1.reshape(1, -1), W2, b2.reshape(1, -1),
      W3, b3.reshape(1, -1), Wp, bp.reshape(1, 1))


def kernel(u, i, gmf_user_table, gmf_item_table, mlp_user_table, mlp_item_table,
           W1, b1, W2, b2, W3, b3, Wp, bp):
    u = u.astype(jnp.int32)
    i = i.astype(jnp.int32)
    gather = _sc_gather()
    packed = [
        jnp.reshape(t, (N // 2, 128))
        for t in (gmf_user_table, gmf_item_table, mlp_user_table, mlp_item_table)
    ]
    qu = u >> 1
    qi = i >> 1
    gu = gather(qu, packed[0])
    gi = gather(qi, packed[1])
    mu = gather(qu, packed[2])
    mi = gather(qi, packed[3])
    upar = (u & 1).reshape(B, 1)
    ipar = (i & 1).reshape(B, 1)
    out = _tc_dense(upar, ipar, gu, gi, mu, mi,
                    W1, b1, W2, b2, W3, b3, Wp, bp)
    return out[:, 0]


# speedup vs baseline: 2.0459x; 1.2289x over previous
import functools

import jax
import jax.numpy as jnp
from jax import lax
from jax.experimental import pallas as pl
from jax.experimental.pallas import tpu as pltpu
from jax.experimental.pallas import tpu_sc as plsc

B = 16384
D = 64
N = 1000000

NC = 2
NS = 16
NW = NC * NS
BPW = B // NW
CH = BPW // 2


def _sc_gather_body(idx_hbm, tab, out, idx_v, buf0, buf1, sg0, sg1, so0, so1):
    wid = lax.axis_index("s") * NC + lax.axis_index("c")
    base = wid * BPW
    pltpu.sync_copy(idx_hbm.at[pl.ds(base, BPW)], idx_v)
    g0 = pltpu.async_copy(tab.at[idx_v.at[pl.ds(0, CH)]], buf0, sg0)
    g1 = pltpu.async_copy(tab.at[idx_v.at[pl.ds(CH, CH)]], buf1, sg1)
    g0.wait()
    o0 = pltpu.async_copy(buf0, out.at[pl.ds(base, CH)], so0)
    g1.wait()
    o1 = pltpu.async_copy(buf1, out.at[pl.ds(base + CH, CH)], so1)
    o0.wait()
    o1.wait()


@functools.cache
def _sc_gather():
    mesh = plsc.VectorSubcoreMesh(
        core_axis_name="c", subcore_axis_name="s", num_cores=NC, num_subcores=NS
    )
    return pl.kernel(
        _sc_gather_body,
        out_type=jax.ShapeDtypeStruct((B, 128), jnp.float32),
        mesh=mesh,
        scratch_types=[
            pltpu.VMEM((BPW,), jnp.int32),
            pltpu.VMEM((CH, 128), jnp.float32),
            pltpu.VMEM((CH, 128), jnp.float32),
            pltpu.SemaphoreType.DMA,
            pltpu.SemaphoreType.DMA,
            pltpu.SemaphoreType.DMA,
            pltpu.SemaphoreType.DMA,
        ],
    )


def _tc_body(su_ref, si_ref,
             w1_ref, b1_ref, w2_ref, b2_ref, w3_ref, b3_ref,
             wp_ref, bp_ref, out_ref):
    su = su_ref[...]
    si = si_ref[...]
    gu, mu = su[:, :D], su[:, D:]
    gi, mi = si[:, :D], si[:, D:]
    w1 = w1_ref[...]
    h = jnp.dot(mu, w1[:D], preferred_element_type=jnp.float32)
    h = h + jnp.dot(mi, w1[D:], preferred_element_type=jnp.float32)
    h = jnp.maximum(h + b1_ref[...], 0.0)
    h = jnp.maximum(
        jnp.dot(h, w2_ref[...], preferred_element_type=jnp.float32) + b2_ref[...], 0.0)
    h = jnp.maximum(
        jnp.dot(h, w3_ref[...], preferred_element_type=jnp.float32) + b3_ref[...], 0.0)
    g = gu * gi
    wp = wp_ref[...]
    pred = jnp.dot(g, wp[:D], preferred_element_type=jnp.float32)
    pred = pred + jnp.dot(h, wp[D:], preferred_element_type=jnp.float32)
    out_ref[...] = pred + bp_ref[...]


def _tc_dense(su, si, W1, b1, W2, b2, W3, b3, Wp, bp):
    R = 2048
    grid = (B // R,)
    row_spec = pl.BlockSpec((R, 128), lambda r: (r, 0))

    def full(shape):
        return pl.BlockSpec(shape, lambda r: (0,) * len(shape))

    return pl.pallas_call(
        _tc_body,
        grid=grid,
        in_specs=[
            row_spec, row_spec,
            full(W1.shape), full((1, b1.shape[0])),
            full(W2.shape), full((1, b2.shape[0])),
            full(W3.shape), full((1, b3.shape[0])),
            full(Wp.shape), full((1, 1)),
        ],
        out_specs=pl.BlockSpec((R, 1), lambda r: (r, 0)),
        out_shape=jax.ShapeDtypeStruct((B, 1), jnp.float32),
    )(su, si, W1, b1.reshape(1, -1), W2, b2.reshape(1, -1),
      W3, b3.reshape(1, -1), Wp, bp.reshape(1, 1))


def kernel(u, i, gmf_user_table, gmf_item_table, mlp_user_table, mlp_item_table,
           W1, b1, W2, b2, W3, b3, Wp, bp):
    u = u.astype(jnp.int32)
    i = i.astype(jnp.int32)
    gather = _sc_gather()
    tu = jnp.concatenate([gmf_user_table, mlp_user_table], axis=1)
    ti = jnp.concatenate([gmf_item_table, mlp_item_table], axis=1)
    su = gather(u, tu)
    si = gather(i, ti)
    out = _tc_dense(su, si, W1, b1, W2, b2, W3, b3, Wp, bp)
    return out[:, 0]
